# Initial kernel scaffold; baseline (speedup 1.0000x reference)
import functools

import jax
import jax.numpy as jnp
from jax import lax
from jax.experimental import pallas as pl
from jax.experimental.pallas import tpu as pltpu
from jax.experimental.pallas import tpu_sc as plsc

BK = 128
BM = 1000


def _sc_mesh():
    return plsc.VectorSubcoreMesh(core_axis_name="c", subcore_axis_name="s")


def _make_deg_kernel(mesh, NS, NBLK, NPAD, RPT):
    def body(dst_hbm, zdeg_hbm, ones_hbm, out_hbm, idx_v, ones_v, deg_sh):
        c = lax.axis_index("c")
        s = lax.axis_index("s")
        wid = c * NS + s
        pltpu.sync_copy(dst_hbm.at[wid], idx_v)
        pltpu.sync_copy(ones_hbm, ones_v)
        pltpu.sync_copy(zdeg_hbm.at[pl.ds(s * RPT, RPT)],
                        deg_sh.at[pl.ds(s * RPT, RPT)])
        plsc.subcore_barrier()

        def blk(j, carry):
            pltpu.sync_copy(ones_v, deg_sh.at[idx_v.at[j]], add=True)
            return carry

        lax.fori_loop(0, NBLK, blk, 0)
        plsc.subcore_barrier()
        pltpu.sync_copy(deg_sh.at[pl.ds(s * RPT, RPT)],
                        out_hbm.at[c, pl.ds(s * RPT, RPT)])

    return pl.kernel(
        body,
        out_type=jax.ShapeDtypeStruct((2, NPAD, 16), jnp.float32),
        mesh=mesh,
        scratch_types=[
            pltpu.VMEM((NBLK, BK), jnp.int32),
            pltpu.VMEM((BK, 16), jnp.float32),
            pltpu.VMEM_SHARED((NPAD, 16), jnp.float32),
        ],
    )


def _make_agg_kernel(mesh, NS, NBLK, NPAD, RPT, D):
    def body(g_hbm, src_hbm, dst_hbm, zrow_hbm, out_hbm,
             sidx_v, didx_v, buf_v, acc_sh, sem):
        c = lax.axis_index("c")
        s = lax.axis_index("s")
        wid = c * NS + s
        pltpu.sync_copy(src_hbm.at[wid], sidx_v)
        pltpu.sync_copy(dst_hbm.at[wid], didx_v)
        pltpu.sync_copy(zrow_hbm.at[pl.ds(s * RPT, RPT)],
                        acc_sh.at[pl.ds(s * RPT, RPT)])
        plsc.subcore_barrier()

        pltpu.async_copy(g_hbm.at[sidx_v.at[0]], buf_v.at[0], sem).wait()

        def blk(j, carry):
            slot = lax.rem(j, 2)
            nxt = lax.rem(j + 1, 2)

            @pl.when(j + 1 < NBLK)
            def _():
                pltpu.async_copy(g_hbm.at[sidx_v.at[j + 1]], buf_v.at[nxt],
                                 sem).start()

            pltpu.sync_copy(buf_v.at[slot], acc_sh.at[didx_v.at[j]], add=True)

            @pl.when(j + 1 < NBLK)
            def _():
                pltpu.make_async_copy(g_hbm.at[sidx_v.at[j + 1]], buf_v.at[nxt],
                                      sem).wait()

            return carry

        lax.fori_loop(0, NBLK, blk, 0)
        plsc.subcore_barrier()
        pltpu.sync_copy(acc_sh.at[pl.ds(s * RPT, RPT)],
                        out_hbm.at[c, pl.ds(s * RPT, RPT)])

    return pl.kernel(
        body,
        out_type=jax.ShapeDtypeStruct((2, NPAD, D), jnp.float32),
        mesh=mesh,
        scratch_types=[
            pltpu.VMEM((NBLK, BK), jnp.int32),
            pltpu.VMEM((NBLK, BK), jnp.int32),
            pltpu.VMEM((2, BK, D), jnp.float32),
            pltpu.VMEM_SHARED((NPAD, D), jnp.float32),
            pltpu.SemaphoreType.DMA,
        ],
    )


def _dis_block(degp_ref):
    deg = degp_ref[0, :, :] + degp_ref[1, :, :] + 1.0
    return lax.rsqrt(jnp.maximum(deg, 1.0))[:, 0:1]


def _tc1_body(x_ref, w_ref, degp_ref, g_ref):
    dis = _dis_block(degp_ref)
    h = jnp.dot(x_ref[...], w_ref[...],
                preferred_element_type=jnp.float32,
                precision=lax.Precision.HIGHEST)
    g_ref[...] = h * dis


def _tc2_body(accp_ref, g1_ref, degp_ref, w_ref, b_ref, g2_ref):
    dis = _dis_block(degp_ref)
    pre = (accp_ref[0] + accp_ref[1] + g1_ref[...]) * dis + b_ref[...]
    h1 = jnp.maximum(pre, 0.0)
    g2_ref[...] = jnp.dot(h1, w_ref[...],
                          preferred_element_type=jnp.float32,
                          precision=lax.Precision.HIGHEST) * dis


def _tc3_body(accp_ref, g2_ref, degp_ref, wd_ref, b2_ref, bd_ref, out_ref):
    dis = _dis_block(degp_ref)
    pre = (accp_ref[0] + accp_ref[1] + g2_ref[...]) * dis + b2_ref[...]
    h2 = jnp.maximum(pre, 0.0)
    out_ref[...] = jnp.dot(h2, wd_ref[...],
                           preferred_element_type=jnp.float32,
                           precision=lax.Precision.HIGHEST) + bd_ref[...]


def kernel(x, edge_index, W1, b1, W2, b2, Wd, bd):
    N, D = x.shape
    E = edge_index.shape[1]
    mesh = _sc_mesh()
    NC, NS = mesh.num_cores, mesh.num_subcores
    NW = NC * NS
    NBLK = -(-E // (NW * BK))
    EPAD = NW * NBLK * BK
    NPAD = -(-(N + 1) // NS) * NS
    RPT = NPAD // NS

    src = edge_index[0]
    dst = edge_index[1]
    pad = EPAD - E
    src_p = jnp.concatenate(
        [src, jnp.zeros((pad,), jnp.int32)]).reshape(NW, NBLK, BK)
    dst_p = jnp.concatenate(
        [dst, jnp.full((pad,), N, jnp.int32)]).reshape(NW, NBLK, BK)
    zdeg = jnp.zeros((NPAD, 16), jnp.float32)
    zrow = jnp.zeros((NPAD, D), jnp.float32)
    ones16 = jnp.ones((BK, 16), jnp.float32)

    deg_k = _make_deg_kernel(mesh, NS, NBLK, NPAD, RPT)
    agg_k = _make_agg_kernel(mesh, NS, NBLK, NPAD, RPT, D)

    degp = deg_k(dst_p, zdeg, ones16)

    grid = N // BM
    wspec = pl.BlockSpec((D, D), lambda i: (0, 0))
    rspec = pl.BlockSpec((BM, D), lambda i: (i, 0))
    dspec = pl.BlockSpec((2, BM, 16), lambda i: (0, i, 0))
    aspec = pl.BlockSpec((2, BM, D), lambda i: (0, i, 0))
    bspec = pl.BlockSpec((1, D), lambda i: (0, 0))
    rout = jax.ShapeDtypeStruct((N, D), jnp.float32)

    g1 = pl.pallas_call(
        _tc1_body, grid=(grid,),
        in_specs=[rspec, wspec, dspec],
        out_specs=rspec, out_shape=rout,
    )(x, W1, degp)

    acc1 = agg_k(g1, src_p, dst_p, zrow)

    g2 = pl.pallas_call(
        _tc2_body, grid=(grid,),
        in_specs=[aspec, rspec, dspec, wspec, bspec],
        out_specs=rspec, out_shape=rout,
    )(acc1, g1, degp, W2, b1.reshape(1, D))

    acc2 = agg_k(g2, src_p, dst_p, zrow)

    out = pl.pallas_call(
        _tc3_body, grid=(grid,),
        in_specs=[aspec, rspec, dspec, wspec, bspec, bspec],
        out_specs=rspec, out_shape=rout,
    )(acc2, g2, degp, Wd, b2.reshape(1, D), bd.reshape(1, D))

    return out


# speedup vs baseline: 12.5714x; 12.5714x over previous
import functools

import jax
import jax.numpy as jnp
from jax import lax
from jax.experimental import pallas as pl
from jax.experimental.pallas import tpu as pltpu
from jax.experimental.pallas import tpu_sc as plsc

BK = 128
BM = 1000


def _sc_mesh():
    return plsc.VectorSubcoreMesh(core_axis_name="c", subcore_axis_name="s")


def _make_deg_kernel(mesh, NS, NBLK, NPAD, RPT, D):
    def body(dst_hbm, zdeg_hbm, ones_hbm, out_hbm, idx_v, ones_v, deg_sh):
        c = lax.axis_index("c")
        s = lax.axis_index("s")
        wid = c * NS + s
        pltpu.sync_copy(dst_hbm.at[wid], idx_v)
        pltpu.sync_copy(ones_hbm, ones_v)
        pltpu.sync_copy(zdeg_hbm.at[pl.ds(s * RPT, RPT)],
                        deg_sh.at[pl.ds(s * RPT, RPT)])
        plsc.subcore_barrier()

        def blk(j, carry):
            pltpu.sync_copy(ones_v, deg_sh.at[idx_v.at[j]], add=True)
            return carry

        lax.fori_loop(0, NBLK, blk, 0)
        plsc.subcore_barrier()
        pltpu.sync_copy(deg_sh.at[pl.ds(s * RPT, RPT)],
                        out_hbm.at[c, pl.ds(s * RPT, RPT)])

    return pl.kernel(
        body,
        out_type=jax.ShapeDtypeStruct((2, NPAD, D), jnp.float32),
        mesh=mesh,
        scratch_types=[
            pltpu.VMEM((NBLK, BK), jnp.int32),
            pltpu.VMEM((BK, D), jnp.float32),
            pltpu.VMEM_SHARED((NPAD, D), jnp.float32),
        ],
    )


def _make_agg_kernel(mesh, NS, NBLK, NPAD, RPT, D):
    def body(g_hbm, src_hbm, dst_hbm, zrow_hbm, out_hbm,
             sidx_v, didx_v, buf_v, acc_sh, sem):
        c = lax.axis_index("c")
        s = lax.axis_index("s")
        wid = c * NS + s
        pltpu.sync_copy(src_hbm.at[wid], sidx_v)
        pltpu.sync_copy(dst_hbm.at[wid], didx_v)
        pltpu.sync_copy(zrow_hbm.at[pl.ds(s * RPT, RPT)],
                        acc_sh.at[pl.ds(s * RPT, RPT)])
        plsc.subcore_barrier()

        def blk(j, carry):
            pltpu.async_copy(g_hbm.at[sidx_v.at[j]], buf_v, sem).wait()
            pltpu.sync_copy(buf_v, acc_sh.at[didx_v.at[j]], add=True)
            return carry

        lax.fori_loop(0, NBLK, blk, 0)
        plsc.subcore_barrier()
        pltpu.sync_copy(acc_sh.at[pl.ds(s * RPT, RPT)],
                        out_hbm.at[c, pl.ds(s * RPT, RPT)])

    return pl.kernel(
        body,
        out_type=jax.ShapeDtypeStruct((2, NPAD, D), jnp.float32),
        mesh=mesh,
        scratch_types=[
            pltpu.VMEM((NBLK, BK), jnp.int32),
            pltpu.VMEM((NBLK, BK), jnp.int32),
            pltpu.VMEM((BK, D), jnp.float32),
            pltpu.VMEM_SHARED((NPAD, D), jnp.float32),
            pltpu.SemaphoreType.DMA,
        ],
    )


def _dis_block(degp_ref):
    deg = degp_ref[0, :, :] + degp_ref[1, :, :] + 1.0
    return lax.rsqrt(jnp.maximum(deg, 1.0))[:, 0:1]


def _tc1_body(x_ref, w_ref, degp_ref, g_ref):
    dis = _dis_block(degp_ref)
    h = jnp.dot(x_ref[...], w_ref[...],
                preferred_element_type=jnp.float32,
                precision=lax.Precision.HIGHEST)
    g_ref[...] = h * dis


def _tc2_body(accp_ref, g1_ref, degp_ref, w_ref, b_ref, g2_ref):
    dis = _dis_block(degp_ref)
    pre = (accp_ref[0] + accp_ref[1] + g1_ref[...]) * dis + b_ref[...]
    h1 = jnp.maximum(pre, 0.0)
    g2_ref[...] = jnp.dot(h1, w_ref[...],
                          preferred_element_type=jnp.float32,
                          precision=lax.Precision.HIGHEST) * dis


def _tc3_body(accp_ref, g2_ref, degp_ref, wd_ref, b2_ref, bd_ref, out_ref):
    dis = _dis_block(degp_ref)
    pre = (accp_ref[0] + accp_ref[1] + g2_ref[...]) * dis + b2_ref[...]
    h2 = jnp.maximum(pre, 0.0)
    out_ref[...] = jnp.dot(h2, wd_ref[...],
                           preferred_element_type=jnp.float32,
                           precision=lax.Precision.HIGHEST) + bd_ref[...]


def kernel(x, edge_index, W1, b1, W2, b2, Wd, bd):
    N, D = x.shape
    E = edge_index.shape[1]
    mesh = _sc_mesh()
    NC, NS = mesh.num_cores, mesh.num_subcores
    NW = NC * NS
    NBLK = -(-E // (NW * BK))
    EPAD = NW * NBLK * BK
    NPAD = -(-(N + 1) // (NS * 8)) * (NS * 8)
    RPT = NPAD // NS

    src = edge_index[0]
    dst = edge_index[1]
    pad = EPAD - E
    src_p = jnp.concatenate(
        [src, jnp.zeros((pad,), jnp.int32)]).reshape(NW, NBLK, BK)
    dst_p = jnp.concatenate(
        [dst, jnp.full((pad,), N, jnp.int32)]).reshape(NW, NBLK, BK)
    zrow = jnp.zeros((NPAD, D), jnp.float32)
    ones_rows = jnp.ones((BK, D), jnp.float32)

    deg_k = _make_deg_kernel(mesh, NS, NBLK, NPAD, RPT, D)
    agg_k = _make_agg_kernel(mesh, NS, NBLK, NPAD, RPT, D)

    degp = deg_k(dst_p, zrow, ones_rows)

    grid = N // BM
    wspec = pl.BlockSpec((D, D), lambda i: (0, 0))
    rspec = pl.BlockSpec((BM, D), lambda i: (i, 0))
    dspec = pl.BlockSpec((2, BM, D), lambda i: (0, i, 0))
    aspec = pl.BlockSpec((2, BM, D), lambda i: (0, i, 0))
    bspec = pl.BlockSpec((1, D), lambda i: (0, 0))
    rout = jax.ShapeDtypeStruct((N, D), jnp.float32)

    g1 = pl.pallas_call(
        _tc1_body, grid=(grid,),
        in_specs=[rspec, wspec, dspec],
        out_specs=rspec, out_shape=rout,
    )(x, W1, degp)

    acc1 = agg_k(g1, src_p, dst_p, zrow)

    g2 = pl.pallas_call(
        _tc2_body, grid=(grid,),
        in_specs=[aspec, rspec, dspec, wspec, bspec],
        out_specs=rspec, out_shape=rout,
    )(acc1, g1, degp, W2, b1.reshape(1, D))

    acc2 = agg_k(g2, src_p, dst_p, zrow)

    out = pl.pallas_call(
        _tc3_body, grid=(grid,),
        in_specs=[aspec, rspec, dspec, wspec, bspec, bspec],
        out_specs=rspec, out_shape=rout,
    )(acc2, g2, degp, Wd, b2.reshape(1, D), bd.reshape(1, D))

    return out
